# baseline (speedup 1.0000x reference)
import jax
import jax.numpy as jnp
from jax import lax
from jax.experimental import pallas as pl
from jax.experimental.pallas import tpu as pltpu
from jax.experimental.pallas import tpu_sc as plsc

_B, _H, _W, _C = 4, 224, 224, 96
_HP, _WP = _H // 2, _W // 2
_NW = 32
_TASKS = _B * _HP
_TPW = _TASKS // _NW


def _sc_body(x_hbm, p_hbm, o_hbm, xv, pv, ov):
    wid = lax.axis_index("s") * 2 + lax.axis_index("c")

    def task(i, carry):
        t = wid * _TPW + i
        b = t // _HP
        hp = t % _HP
        pltpu.sync_copy(x_hbm.at[b, pl.ds(2 * hp, 2)], xv)
        pltpu.sync_copy(p_hbm.at[b, hp], pv)

        def col(wp, cc):
            w0 = 2 * wp
            for j in range(_C // 16):
                cj = 16 * j
                v00 = xv[0, w0, pl.ds(cj, 16)]
                v01 = xv[0, w0 + 1, pl.ds(cj, 16)]
                v10 = xv[1, w0, pl.ds(cj, 16)]
                v11 = xv[1, w0 + 1, pl.ds(cj, 16)]
                m = jnp.maximum(jnp.maximum(v00, v01), jnp.maximum(v10, v11))
                r = jnp.maximum(pv[wp, pl.ds(cj, 16)], 0.0)
                z = jnp.zeros_like(r)
                c0 = v00 >= m
                ov[0, w0, pl.ds(cj, 16)] = jnp.where(c0, r, z)
                rr = jnp.where(c0, z, r)
                c1 = v01 >= m
                ov[0, w0 + 1, pl.ds(cj, 16)] = jnp.where(c1, rr, z)
                rr = jnp.where(c1, z, rr)
                c2 = v10 >= m
                ov[1, w0, pl.ds(cj, 16)] = jnp.where(c2, rr, z)
                ov[1, w0 + 1, pl.ds(cj, 16)] = jnp.where(c2, z, rr)
            return cc

        lax.fori_loop(0, _WP, col, None)
        pltpu.sync_copy(ov, o_hbm.at[b, pl.ds(2 * hp, 2)])
        return carry

    lax.fori_loop(0, _TPW, task, None)


def kernel(x, pooled):
    B, H, W, C = x.shape
    mesh = plsc.VectorSubcoreMesh(core_axis_name="c", subcore_axis_name="s", num_cores=2, num_subcores=16)
    return pl.kernel(
        _sc_body,
        out_type=jax.ShapeDtypeStruct((B, H, W, C), x.dtype),
        mesh=mesh,
        scratch_types=[
            pltpu.VMEM((2, _W, _C), jnp.float32),
            pltpu.VMEM((_WP, _C), jnp.float32),
            pltpu.VMEM((2, _W, _C), jnp.float32),
        ],
    )(x, pooled)


# speedup vs baseline: 1.2131x; 1.2131x over previous
import jax
import jax.numpy as jnp
from jax import lax
from jax.experimental import pallas as pl
from jax.experimental.pallas import tpu as pltpu
from jax.experimental.pallas import tpu_sc as plsc

_B, _H, _W, _C = 4, 224, 224, 96
_HP, _WP = _H // 2, _W // 2
_WH, _WPH = _W // 2, _WP // 2
_NW = 32
_TASKS = _B * _HP * 2
_TPW = _TASKS // _NW


def _sc_body(x_hbm, p_hbm, o_hbm, xv, pv, ov, six, sip, sout):
    wid = lax.axis_index("s") * 2 + lax.axis_index("c")
    t0 = wid * _TPW

    def coords(i):
        t = t0 + i
        b = t // (_HP * 2)
        rem = t % (_HP * 2)
        hp = rem // 2
        wh = rem % 2
        return b, hp, wh

    def start_in(i, buf):
        b, hp, wh = coords(i)
        dx = pltpu.async_copy(
            x_hbm.at[b, pl.ds(2 * hp, 2), pl.ds(wh * _WH, _WH)],
            xv.at[buf], six.at[buf])
        dp = pltpu.async_copy(
            p_hbm.at[b, hp, pl.ds(wh * _WPH, _WPH)],
            pv.at[buf], sip.at[buf])
        return dx, dp

    def start_out(i, buf):
        b, hp, wh = coords(i)
        return pltpu.async_copy(
            ov.at[buf],
            o_hbm.at[b, pl.ds(2 * hp, 2), pl.ds(wh * _WH, _WH)], sout.at[buf])

    def compute(buf):
        @plsc.parallel_loop(0, _WPH, unroll=2)
        def col(wp):
            w0 = 2 * wp
            for j in range(_C // 16):
                cj = 16 * j
                v00 = xv[buf, 0, w0, pl.ds(cj, 16)]
                v01 = xv[buf, 0, w0 + 1, pl.ds(cj, 16)]
                v10 = xv[buf, 1, w0, pl.ds(cj, 16)]
                v11 = xv[buf, 1, w0 + 1, pl.ds(cj, 16)]
                m = jnp.maximum(jnp.maximum(v00, v01), jnp.maximum(v10, v11))
                r = jnp.maximum(pv[buf, wp, pl.ds(cj, 16)], 0.0)
                z = jnp.zeros_like(r)
                c0 = v00 >= m
                ov[buf, 0, w0, pl.ds(cj, 16)] = jnp.where(c0, r, z)
                rr = jnp.where(c0, z, r)
                c1 = v01 >= m
                ov[buf, 0, w0 + 1, pl.ds(cj, 16)] = jnp.where(c1, rr, z)
                rr = jnp.where(c1, z, rr)
                c2 = v10 >= m
                ov[buf, 1, w0, pl.ds(cj, 16)] = jnp.where(c2, rr, z)
                ov[buf, 1, w0 + 1, pl.ds(cj, 16)] = jnp.where(c2, z, rr)

    in_flight = {0: start_in(0, 0)}
    out_flight = {}
    for i in range(_TPW):
        cur = i % 2
        if i + 1 < _TPW:
            in_flight[i + 1] = start_in(i + 1, 1 - cur)
        dx, dp = in_flight.pop(i)
        dx.wait()
        dp.wait()
        if i >= 2:
            out_flight.pop(i - 2).wait()
        compute(cur)
        out_flight[i] = start_out(i, cur)
    out_flight.pop(_TPW - 2).wait()
    out_flight.pop(_TPW - 1).wait()


def kernel(x, pooled):
    B, H, W, C = x.shape
    mesh = plsc.VectorSubcoreMesh(core_axis_name="c", subcore_axis_name="s",
                                  num_cores=2, num_subcores=16)
    return pl.kernel(
        _sc_body,
        out_type=jax.ShapeDtypeStruct((B, H, W, C), x.dtype),
        mesh=mesh,
        scratch_types=[
            pltpu.VMEM((2, 2, _WH, _C), jnp.float32),
            pltpu.VMEM((2, _WPH, _C), jnp.float32),
            pltpu.VMEM((2, 2, _WH, _C), jnp.float32),
            pltpu.SemaphoreType.DMA((2,)),
            pltpu.SemaphoreType.DMA((2,)),
            pltpu.SemaphoreType.DMA((2,)),
        ],
    )(x, pooled)


# speedup vs baseline: 1.2135x; 1.0003x over previous
import jax
import jax.numpy as jnp
from jax import lax
from jax.experimental import pallas as pl
from jax.experimental.pallas import tpu as pltpu
from jax.experimental.pallas import tpu_sc as plsc

_B, _H, _W, _C = 4, 224, 224, 96
_HP, _WP = _H // 2, _W // 2
_WH, _WPH = _W // 2, _WP // 2
_NW = 32
_TASKS = _B * _HP * 2
_TPW = _TASKS // _NW


def _sc_body(x_hbm, p_hbm, o_hbm, xv, pv, ov, six, sip, sout):
    wid = lax.axis_index("s") * 2 + lax.axis_index("c")
    t0 = wid * _TPW

    def coords(i):
        t = t0 + i
        b = t // (_HP * 2)
        rem = t % (_HP * 2)
        hp = rem // 2
        wh = rem % 2
        return b, hp, wh

    def start_in(i, buf):
        b, hp, wh = coords(i)
        dx = pltpu.async_copy(
            x_hbm.at[b, pl.ds(2 * hp, 2), pl.ds(wh * _WH, _WH)],
            xv.at[buf], six.at[buf])
        dp = pltpu.async_copy(
            p_hbm.at[b, hp, pl.ds(wh * _WPH, _WPH)],
            pv.at[buf], sip.at[buf])
        return dx, dp

    def start_out(i, buf):
        b, hp, wh = coords(i)
        return pltpu.async_copy(
            ov.at[buf],
            o_hbm.at[b, pl.ds(2 * hp, 2), pl.ds(wh * _WH, _WH)], sout.at[buf])

    def compute(buf):
        @plsc.parallel_loop(0, _WPH, unroll=2)
        def col(wp):
            w0 = 2 * wp
            for j in range(_C // 16):
                cj = 16 * j
                v00 = xv[buf, 0, w0, pl.ds(cj, 16)]
                v01 = xv[buf, 0, w0 + 1, pl.ds(cj, 16)]
                v10 = xv[buf, 1, w0, pl.ds(cj, 16)]
                v11 = xv[buf, 1, w0 + 1, pl.ds(cj, 16)]
                m = jnp.maximum(jnp.maximum(v00, v01), jnp.maximum(v10, v11))
                r = jnp.maximum(pv[buf, wp, pl.ds(cj, 16)], 0.0)
                z = jnp.zeros_like(r)
                c0 = v00 >= m
                ov[buf, 0, w0, pl.ds(cj, 16)] = jnp.where(c0, r, z)
                rr = jnp.where(c0, z, r)
                c1 = v01 >= m
                ov[buf, 0, w0 + 1, pl.ds(cj, 16)] = jnp.where(c1, rr, z)
                rr = jnp.where(c1, z, rr)
                c2 = v10 >= m
                ov[buf, 1, w0, pl.ds(cj, 16)] = jnp.where(c2, rr, z)
                ov[buf, 1, w0 + 1, pl.ds(cj, 16)] = jnp.where(c2, z, rr)

    in_flight = {0: start_in(0, 0)}
    out_flight = {}
    for i in range(_TPW):
        cur = i % 2
        if i + 1 < _TPW:
            in_flight[i + 1] = start_in(i + 1, 1 - cur)
        dx, dp = in_flight.pop(i)
        dx.wait()
        dp.wait()
        if i >= 2:
            out_flight.pop(i - 2).wait()
        compute(cur)
        out_flight[i] = start_out(i, cur)
    out_flight.pop(_TPW - 2).wait()
    out_flight.pop(_TPW - 1).wait()


def kernel(x, pooled):
    B, H, W, C = x.shape
    mesh = plsc.VectorSubcoreMesh(core_axis_name="c", subcore_axis_name="s",
                                  num_cores=2, num_subcores=16)
    return pl.kernel(
        _sc_body,
        out_type=jax.ShapeDtypeStruct((B, H, W, C), x.dtype),
        mesh=mesh,
        compiler_params=pltpu.CompilerParams(use_tc_tiling_on_sc=True),
        scratch_types=[
            pltpu.VMEM((2, 2, _WH, _C), jnp.float32),
            pltpu.VMEM((2, _WPH, _C), jnp.float32),
            pltpu.VMEM((2, 2, _WH, _C), jnp.float32),
            pltpu.SemaphoreType.DMA((2,)),
            pltpu.SemaphoreType.DMA((2,)),
            pltpu.SemaphoreType.DMA((2,)),
        ],
    )(x, pooled)


# speedup vs baseline: 3.6000x; 2.9667x over previous
import jax
import jax.numpy as jnp
from jax import lax
from jax.experimental import pallas as pl
from jax.experimental.pallas import tpu as pltpu
from jax.experimental.pallas import tpu_sc as plsc

_B, _H, _W, _C = 4, 224, 224, 96
_HP, _WP = _H // 2, _W // 2
_CH = _C // 2
_NW = 32
_TASKS = _B * _HP * 2
_TPW = _TASKS // _NW


def _sc_body(x_hbm, p_hbm, o_hbm, xv, pv, ov, six, sip, sout):
    wid = lax.axis_index("s") * 2 + lax.axis_index("c")
    t0 = wid * _TPW
    idx_e = lax.iota(jnp.int32, 16) * 2
    idx_o = idx_e + 1

    def coords(i):
        t = t0 + i
        b = t // (_HP * 2)
        rem = t % (_HP * 2)
        hp = rem // 2
        ch = rem % 2
        return b, hp, ch

    def start_in(i, buf):
        b, hp, ch = coords(i)
        dx = pltpu.async_copy(
            x_hbm.at[b, pl.ds(2 * hp, 2), pl.ds(ch * _CH, _CH)],
            xv.at[buf], six.at[buf])
        dp = pltpu.async_copy(
            p_hbm.at[b, hp, pl.ds(ch * _CH, _CH)],
            pv.at[buf], sip.at[buf])
        return dx, dp

    def start_out(i, buf):
        b, hp, ch = coords(i)
        return pltpu.async_copy(
            ov.at[buf],
            o_hbm.at[b, pl.ds(2 * hp, 2), pl.ds(ch * _CH, _CH)],
            sout.at[buf])

    def compute(buf):
        ib = jnp.broadcast_to(jnp.int32(buf), (16,))
        i0 = jnp.broadcast_to(jnp.int32(0), (16,))
        i1 = jnp.broadcast_to(jnp.int32(1), (16,))

        @plsc.parallel_loop(0, _CH, unroll=1)
        def chan(c):
            ic = jnp.broadcast_to(c, (16,))
            for k in range(_W // 32):
                ie = idx_e + (32 * k)
                io = ie + 1
                v00 = plsc.load_gather(xv, [ib, i0, ic, ie])
                v01 = plsc.load_gather(xv, [ib, i0, ic, io])
                v10 = plsc.load_gather(xv, [ib, i1, ic, ie])
                v11 = plsc.load_gather(xv, [ib, i1, ic, io])
                m = jnp.maximum(jnp.maximum(v00, v01), jnp.maximum(v10, v11))
                r = jnp.maximum(pv[buf, c, pl.ds(16 * k, 16)], 0.0)
                z = jnp.zeros_like(r)
                c0 = v00 >= m
                plsc.store_scatter(ov, [ib, i0, ic, ie], jnp.where(c0, r, z))
                rr = jnp.where(c0, z, r)
                c1 = v01 >= m
                plsc.store_scatter(ov, [ib, i0, ic, io], jnp.where(c1, rr, z))
                rr = jnp.where(c1, z, rr)
                c2 = v10 >= m
                plsc.store_scatter(ov, [ib, i1, ic, ie], jnp.where(c2, rr, z))
                plsc.store_scatter(ov, [ib, i1, ic, io], jnp.where(c2, z, rr))

    in_flight = {0: start_in(0, 0)}
    out_flight = {}
    for i in range(_TPW):
        cur = i % 2
        if i + 1 < _TPW:
            in_flight[i + 1] = start_in(i + 1, 1 - cur)
        dx, dp = in_flight.pop(i)
        dx.wait()
        dp.wait()
        if i >= 2:
            out_flight.pop(i - 2).wait()
        compute(cur)
        out_flight[i] = start_out(i, cur)
    out_flight.pop(_TPW - 2).wait()
    out_flight.pop(_TPW - 1).wait()


def kernel(x, pooled):
    B, H, W, C = x.shape
    xt = jnp.swapaxes(x, 2, 3)
    pt = jnp.swapaxes(pooled, 2, 3)
    mesh = plsc.VectorSubcoreMesh(core_axis_name="c", subcore_axis_name="s",
                                  num_cores=2, num_subcores=16)
    out_t = pl.kernel(
        _sc_body,
        out_type=jax.ShapeDtypeStruct((B, H, C, W), x.dtype),
        mesh=mesh,
        compiler_params=pltpu.CompilerParams(needs_layout_passes=False),
        scratch_types=[
            pltpu.VMEM((2, 2, _CH, _W), jnp.float32),
            pltpu.VMEM((2, _CH, _WP), jnp.float32),
            pltpu.VMEM((2, 2, _CH, _W), jnp.float32),
            pltpu.SemaphoreType.DMA((2,)),
            pltpu.SemaphoreType.DMA((2,)),
            pltpu.SemaphoreType.DMA((2,)),
        ],
    )(xt, pt)
    return jnp.swapaxes(out_t, 2, 3)
